# baseline (speedup 1.0000x reference)
import functools

import jax
import jax.numpy as jnp
from jax import lax
from jax.experimental import pallas as pl
from jax.experimental.pallas import tpu as pltpu
from jax.experimental.pallas import tpu_sc as plsc

D = 128
L = 50
V = 100000
VP = 102400
M = 1000
MP = 1024
C = 20
P = 5
NW = 32
KSEG = 32
KIDX = KSEG * 64
VTOK = 1664
NROW = 24

_mesh = plsc.VectorSubcoreMesh(core_axis_name="c", subcore_axis_name="s")


@functools.partial(
    pl.kernel,
    out_type=jax.ShapeDtypeStruct((NW, D), jnp.float32),
    mesh=_mesh,
    scratch_types=[
        pltpu.VMEM((64,), jnp.int32),
        pltpu.VMEM((64, D), jnp.float32),
        pltpu.VMEM((1, D), jnp.float32),
        pltpu.SemaphoreType.DMA,
    ],
    compiler_params=pltpu.CompilerParams(needs_layout_passes=False),
)
def _pool_small_sc(emb_hbm, idx_hbm, out_hbm, idx_v, rows_v, out_v, sem):
    cid = lax.axis_index("c")
    sid = lax.axis_index("s")
    wid = sid * 2 + cid
    pltpu.sync_copy(idx_hbm.at[pl.ds(wid * 64, 64)], idx_v)
    for t in range(4):
        iv = idx_v[pl.ds(16 * t, 16)]
        pltpu.async_copy(emb_hbm.at[iv], rows_v.at[pl.ds(16 * t, 16)], sem)
    for t in range(4):
        iv = idx_v[pl.ds(16 * t, 16)]
        pltpu.make_async_copy(
            emb_hbm.at[iv], rows_v.at[pl.ds(16 * t, 16)], sem).wait()

    def row_body(r, acc):
        return tuple(acc[k] + rows_v[r, pl.ds(16 * k, 16)] for k in range(8))

    acc = lax.fori_loop(0, L, row_body,
                        tuple(jnp.zeros((16,), jnp.float32) for _ in range(8)))
    for k in range(8):
        out_v[0, pl.ds(16 * k, 16)] = acc[k] * (1.0 / L)
    pltpu.sync_copy(out_v, out_hbm.at[pl.ds(wid, 1)])


def _qg_tc(pooled_ref, rw_ref, out_ref):
    pooled = pooled_ref[...]
    rw = rw_ref[...]
    enc_cands = pooled[0:C]
    enc_persona = pooled[C:C + P]
    enc_x = pooled[C + P:C + P + 1]
    eps = 1e-6
    dot = jnp.sum(enc_x * enc_persona, axis=1, keepdims=True)
    na = jnp.sqrt(jnp.sum(enc_x * enc_x, axis=1, keepdims=True))
    nb = jnp.sqrt(jnp.sum(enc_persona * enc_persona, axis=1, keepdims=True))
    sim = dot / (jnp.maximum(na, eps) * jnp.maximum(nb, eps))
    m = jnp.max(sim, axis=0, keepdims=True)
    ex = jnp.exp(sim - m)
    ss = ex / jnp.sum(ex, axis=0, keepdims=True)
    test = jnp.dot(ss.T, enc_persona, preferred_element_type=jnp.float32)
    q = jnp.dot(test, rw.T, preferred_element_type=jnp.float32)
    g = jnp.dot(enc_cands, rw, preferred_element_type=jnp.float32)
    out_ref[0:1, :] = q
    out_ref[1:1 + C, :] = g
    out_ref[1 + C:, :] = jnp.zeros((NROW - 1 - C, D), jnp.float32)


_CBLK = 4096

def _table_tc(pooled_ref, rw_ref, emb_ref, y_ref, wall_ref):
    @pl.when(pl.program_id(0) == 0)
    def _():
        _qg_tc(pooled_ref, rw_ref, wall_ref)

    y_ref[...] = jax.lax.dot_general(
        wall_ref[...], emb_ref[...],
        dimension_numbers=(((1,), (1,)), ((), ())),
        preferred_element_type=jnp.float32)


@functools.partial(
    pl.kernel,
    out_type=(jax.ShapeDtypeStruct((MP,), jnp.float32),
              jax.ShapeDtypeStruct((2, VP), jnp.float32)),
    mesh=_mesh,
    scratch_types=[
        pltpu.VMEM((16, 128), jnp.int32),
        pltpu.VMEM((2048,), jnp.float32),
        pltpu.VMEM((48,), jnp.float32),
        pltpu.VMEM((13, 128), jnp.int32),
        pltpu.VMEM((13, 128), jnp.int32),
        pltpu.VMEM((13, 128), jnp.float32),
        pltpu.VMEM_SHARED((VP,), jnp.float32),
        pltpu.SemaphoreType.DMA,
    ],
    compiler_params=pltpu.CompilerParams(needs_layout_passes=False),
)
def _kv_sc(yq_hbm, kidx_hbm, vidx_hbm, smap_hbm, zeros_hbm,
           e_hbm, w_hbm, kidx_v, kval_v, e_v, vidx_v, smap_v, u_v, w_sp, sem):
    cid = lax.axis_index("c")
    sid = lax.axis_index("s")
    wid = sid * 2 + cid

    pltpu.sync_copy(kidx_hbm.at[wid], kidx_v)
    for j in range(16):
        pltpu.async_copy(yq_hbm.at[kidx_v.at[j]],
                         kval_v.at[pl.ds(128 * j, 128)], sem)
    for j in range(16):
        pltpu.make_async_copy(yq_hbm.at[kidx_v.at[j]],
                              kval_v.at[pl.ds(128 * j, 128)], sem).wait()

    lanes = lax.iota(jnp.int32, 16)
    for g in range(2):
        base = lanes * 64 + g * 1024
        ssum = jnp.zeros((16,), jnp.float32)
        for t in range(L):
            ssum = ssum + plsc.load_gather(kval_v, [base + t])
        seg_global = wid * KSEG + g * 16 + lanes
        e = jnp.exp(ssum * (1.0 / L))
        e = jnp.where(seg_global < M, e, 0.0)
        e_v[pl.ds(g * 16, 16)] = e
    e_v[pl.ds(32, 16)] = jnp.zeros((16,), jnp.float32)
    pltpu.sync_copy(e_v.at[pl.ds(0, KSEG)], e_hbm.at[pl.ds(wid * KSEG, KSEG)])

    pltpu.sync_copy(vidx_hbm.at[wid], vidx_v)
    pltpu.sync_copy(smap_hbm, smap_v)
    for j in range(13):
        for t in range(8):
            sm = smap_v[j, pl.ds(16 * t, 16)]
            u_v[j, pl.ds(16 * t, 16)] = plsc.load_gather(e_v, [sm])

    wslc = VP // 16
    pltpu.sync_copy(zeros_hbm.at[pl.ds(sid * wslc, wslc)],
                    w_sp.at[pl.ds(sid * wslc, wslc)])
    plsc.subcore_barrier()
    for j in range(13):
        pltpu.sync_copy(u_v.at[j], w_sp.at[vidx_v.at[j]], add=True)
    plsc.subcore_barrier()
    pltpu.sync_copy(w_sp.at[pl.ds(sid * wslc, wslc)],
                    w_hbm.at[cid, pl.ds(sid * wslc, wslc)])


_EBLK = 6400

def _logits_tc(y_ref, w_ref, e_ref, out_ref, acc_ref):
    i = pl.program_id(0)

    @pl.when(i == 0)
    def _():
        acc_ref[...] = jnp.zeros((NROW, 1), jnp.float32)

    ws = w_ref[0:1, :] + w_ref[1:2, :]
    col = i * _EBLK + jax.lax.broadcasted_iota(jnp.int32, (1, _EBLK), 1)
    yblk = jnp.where(col < V, y_ref[...], 0.0)
    acc_ref[...] += jax.lax.dot_general(
        yblk, ws,
        dimension_numbers=(((1,), (1,)), ((), ())),
        preferred_element_type=jnp.float32)

    @pl.when(i == pl.num_programs(0) - 1)
    def _():
        z = jnp.sum(e_ref[...])
        logits = acc_ref[1:1 + C, :] * (1.0 / (L * z))
        mx = jnp.max(logits, axis=0, keepdims=True)
        ex = jnp.exp(logits - mx)
        out_ref[...] = ex / jnp.sum(ex, axis=0, keepdims=True)


def kernel(xs, candidates, persona, label, keys, values, emb_table, R_W):
    del label
    emb = emb_table.astype(jnp.float32)
    rw = R_W.astype(jnp.float32)

    small = jnp.concatenate([
        candidates.reshape(-1), persona.reshape(-1), xs.reshape(-1),
    ]).astype(jnp.int32).reshape(C + P + 1, L)
    idx_small = (jnp.zeros((NW, 64), jnp.int32)
                 .at[:C + P + 1, :L].set(small).reshape(-1))
    pooled_small = _pool_small_sc(emb, idx_small)

    y = pl.pallas_call(
        _table_tc,
        grid=(VP // _CBLK,),
        in_specs=[
            pl.BlockSpec((NW, D), lambda i: (0, 0)),
            pl.BlockSpec((D, D), lambda i: (0, 0)),
            pl.BlockSpec((_CBLK, D), lambda i: (i, 0)),
        ],
        out_specs=pl.BlockSpec((NROW, _CBLK), lambda i: (0, i)),
        out_shape=jax.ShapeDtypeStruct((NROW, VP), jnp.float32),
        scratch_shapes=[pltpu.VMEM((NROW, D), jnp.float32)],
    )(pooled_small, rw, emb)

    yq = y[0]

    kidx = (jnp.zeros((MP, 64), jnp.int32)
            .at[:M, :L].set(keys.astype(jnp.int32))
            .reshape(NW, 16, 128))
    vpad = jnp.zeros((MP * L,), jnp.int32).at[:M * L].set(
        values.astype(jnp.int32).reshape(-1))
    vidx = (jnp.zeros((NW, VTOK), jnp.int32)
            .at[:, :KSEG * L].set(vpad.reshape(NW, KSEG * L))
            .reshape(NW, 13, 128))
    i = jnp.arange(VTOK, dtype=jnp.int32)
    smap = jnp.where(i < KSEG * L, i // L, KSEG).astype(
        jnp.int32).reshape(13, 128)
    zeros = jnp.zeros((VP,), jnp.float32)
    e, w2 = _kv_sc(yq, kidx, vidx, smap, zeros)

    preds = pl.pallas_call(
        _logits_tc,
        grid=(VP // _EBLK,),
        in_specs=[
            pl.BlockSpec((NROW, _EBLK), lambda i: (0, i)),
            pl.BlockSpec((2, _EBLK), lambda i: (0, i)),
            pl.BlockSpec((8, 128), lambda i: (0, 0)),
        ],
        out_specs=pl.BlockSpec((C, 1), lambda i: (0, 0)),
        out_shape=jax.ShapeDtypeStruct((C, 1), jnp.float32),
        scratch_shapes=[pltpu.VMEM((NROW, 1), jnp.float32)],
    )(y, w2, e.reshape(8, 128))
    return preds


# speedup vs baseline: 1.0028x; 1.0028x over previous
import functools

import jax
import jax.numpy as jnp
from jax import lax
from jax.experimental import pallas as pl
from jax.experimental.pallas import tpu as pltpu
from jax.experimental.pallas import tpu_sc as plsc

D = 128
L = 50
V = 100000
VP = 102400
M = 1000
MP = 1024
C = 20
P = 5
NW = 32
KSEG = 32
KIDX = KSEG * 64
VTOK = 1664
NROW = 24

_mesh = plsc.VectorSubcoreMesh(core_axis_name="c", subcore_axis_name="s")


@functools.partial(
    pl.kernel,
    out_type=jax.ShapeDtypeStruct((NW, D), jnp.float32),
    mesh=_mesh,
    scratch_types=[
        pltpu.VMEM((64,), jnp.int32),
        pltpu.VMEM((64, D), jnp.float32),
        pltpu.VMEM((1, D), jnp.float32),
        pltpu.SemaphoreType.DMA,
    ],
    compiler_params=pltpu.CompilerParams(needs_layout_passes=False),
)
def _pool_small_sc(emb_hbm, idx_hbm, out_hbm, idx_v, rows_v, out_v, sem):
    cid = lax.axis_index("c")
    sid = lax.axis_index("s")
    wid = sid * 2 + cid
    pltpu.sync_copy(idx_hbm.at[pl.ds(wid * 64, 64)], idx_v)
    for t in range(4):
        iv = idx_v[pl.ds(16 * t, 16)]
        pltpu.async_copy(emb_hbm.at[iv], rows_v.at[pl.ds(16 * t, 16)], sem)
    for t in range(4):
        iv = idx_v[pl.ds(16 * t, 16)]
        pltpu.make_async_copy(
            emb_hbm.at[iv], rows_v.at[pl.ds(16 * t, 16)], sem).wait()

    def row_body(r, acc):
        return tuple(acc[k] + rows_v[r, pl.ds(16 * k, 16)] for k in range(8))

    acc = lax.fori_loop(0, L, row_body,
                        tuple(jnp.zeros((16,), jnp.float32) for _ in range(8)))
    for k in range(8):
        out_v[0, pl.ds(16 * k, 16)] = acc[k] * (1.0 / L)
    pltpu.sync_copy(out_v, out_hbm.at[pl.ds(wid, 1)])


def _qg_tc(pooled_ref, rw_ref, out_ref):
    pooled = pooled_ref[...]
    rw = rw_ref[...]
    enc_cands = pooled[0:C]
    enc_persona = pooled[C:C + P]
    enc_x = pooled[C + P:C + P + 1]
    eps = 1e-6
    dot = jnp.sum(enc_x * enc_persona, axis=1, keepdims=True)
    na = jnp.sqrt(jnp.sum(enc_x * enc_x, axis=1, keepdims=True))
    nb = jnp.sqrt(jnp.sum(enc_persona * enc_persona, axis=1, keepdims=True))
    sim = dot / (jnp.maximum(na, eps) * jnp.maximum(nb, eps))
    m = jnp.max(sim, axis=0, keepdims=True)
    ex = jnp.exp(sim - m)
    ss = ex / jnp.sum(ex, axis=0, keepdims=True)
    test = jnp.dot(ss.T, enc_persona, preferred_element_type=jnp.float32)
    q = jnp.dot(test, rw.T, preferred_element_type=jnp.float32)
    g = jnp.dot(enc_cands, rw, preferred_element_type=jnp.float32)
    out_ref[0:1, :] = q
    out_ref[1:1 + C, :] = g
    out_ref[1 + C:, :] = jnp.zeros((NROW - 1 - C, D), jnp.float32)


_CBLK = 4096

def _table_tc(pooled_ref, rw_ref, emb_ref, y_ref, wall_ref):
    @pl.when(pl.program_id(0) == 0)
    def _():
        _qg_tc(pooled_ref, rw_ref, wall_ref)

    y_ref[...] = jax.lax.dot_general(
        wall_ref[...], emb_ref[...],
        dimension_numbers=(((1,), (1,)), ((), ())),
        preferred_element_type=jnp.float32)


@functools.partial(
    pl.kernel,
    out_type=(jax.ShapeDtypeStruct((MP,), jnp.float32),
              jax.ShapeDtypeStruct((2, VP), jnp.float32)),
    mesh=_mesh,
    scratch_types=[
        pltpu.VMEM((16, 128), jnp.int32),
        pltpu.VMEM((2048,), jnp.float32),
        pltpu.VMEM((48,), jnp.float32),
        pltpu.VMEM((13, 128), jnp.int32),
        pltpu.VMEM((13, 128), jnp.int32),
        pltpu.VMEM((13, 128), jnp.float32),
        pltpu.VMEM_SHARED((VP,), jnp.float32),
        pltpu.SemaphoreType.DMA,
    ],
    compiler_params=pltpu.CompilerParams(needs_layout_passes=False),
)
def _kv_sc(yq_hbm, kidx_hbm, vidx_hbm, smap_hbm, zeros_hbm,
           e_hbm, w_hbm, kidx_v, kval_v, e_v, vidx_v, smap_v, u_v, w_sp, sem):
    cid = lax.axis_index("c")
    sid = lax.axis_index("s")
    wid = sid * 2 + cid

    with jax.named_scope("kv_keys_gather"):
        pltpu.sync_copy(kidx_hbm.at[wid], kidx_v)
        for j in range(16):
            pltpu.async_copy(yq_hbm.at[kidx_v.at[j]],
                             kval_v.at[pl.ds(128 * j, 128)], sem)
        for j in range(16):
            pltpu.make_async_copy(yq_hbm.at[kidx_v.at[j]],
                                  kval_v.at[pl.ds(128 * j, 128)], sem).wait()

    with jax.named_scope("kv_seg_sums"):
        lanes = lax.iota(jnp.int32, 16)
        for g in range(2):
            base = lanes * 64 + g * 1024
            ssum = jnp.zeros((16,), jnp.float32)
            for t in range(L):
                ssum = ssum + plsc.load_gather(kval_v, [base + t])
            seg_global = wid * KSEG + g * 16 + lanes
            e = jnp.exp(ssum * (1.0 / L))
            e = jnp.where(seg_global < M, e, 0.0)
            e_v[pl.ds(g * 16, 16)] = e
        e_v[pl.ds(32, 16)] = jnp.zeros((16,), jnp.float32)
        pltpu.sync_copy(e_v.at[pl.ds(0, KSEG)],
                        e_hbm.at[pl.ds(wid * KSEG, KSEG)])

    with jax.named_scope("kv_u_build"):
        pltpu.sync_copy(vidx_hbm.at[wid], vidx_v)
        pltpu.sync_copy(smap_hbm, smap_v)
        for j in range(13):
            for t in range(8):
                sm = smap_v[j, pl.ds(16 * t, 16)]
                u_v[j, pl.ds(16 * t, 16)] = plsc.load_gather(e_v, [sm])

    wslc = VP // 16
    with jax.named_scope("kv_w_init"):
        pltpu.sync_copy(zeros_hbm.at[pl.ds(sid * wslc, wslc)],
                        w_sp.at[pl.ds(sid * wslc, wslc)])
        plsc.subcore_barrier()
    with jax.named_scope("kv_scatter"):
        for j in range(13):
            pltpu.sync_copy(u_v.at[j], w_sp.at[vidx_v.at[j]], add=True)
        plsc.subcore_barrier()
    with jax.named_scope("kv_w_out"):
        pltpu.sync_copy(w_sp.at[pl.ds(sid * wslc, wslc)],
                        w_hbm.at[cid, pl.ds(sid * wslc, wslc)])


_EBLK = 6400

def _logits_tc(y_ref, w_ref, e_ref, out_ref, acc_ref):
    i = pl.program_id(0)

    @pl.when(i == 0)
    def _():
        acc_ref[...] = jnp.zeros((NROW, 1), jnp.float32)

    ws = w_ref[0:1, :] + w_ref[1:2, :]
    col = i * _EBLK + jax.lax.broadcasted_iota(jnp.int32, (1, _EBLK), 1)
    yblk = jnp.where(col < V, y_ref[...], 0.0)
    acc_ref[...] += jax.lax.dot_general(
        yblk, ws,
        dimension_numbers=(((1,), (1,)), ((), ())),
        preferred_element_type=jnp.float32)

    @pl.when(i == pl.num_programs(0) - 1)
    def _():
        z = jnp.sum(e_ref[...])
        logits = acc_ref[1:1 + C, :] * (1.0 / (L * z))
        mx = jnp.max(logits, axis=0, keepdims=True)
        ex = jnp.exp(logits - mx)
        out_ref[...] = ex / jnp.sum(ex, axis=0, keepdims=True)


def kernel(xs, candidates, persona, label, keys, values, emb_table, R_W):
    del label
    emb = emb_table.astype(jnp.float32)
    rw = R_W.astype(jnp.float32)

    small = jnp.concatenate([
        candidates.reshape(-1), persona.reshape(-1), xs.reshape(-1),
    ]).astype(jnp.int32).reshape(C + P + 1, L)
    idx_small = (jnp.zeros((NW, 64), jnp.int32)
                 .at[:C + P + 1, :L].set(small).reshape(-1))
    pooled_small = _pool_small_sc(emb, idx_small)

    y = pl.pallas_call(
        _table_tc,
        grid=(VP // _CBLK,),
        in_specs=[
            pl.BlockSpec((NW, D), lambda i: (0, 0)),
            pl.BlockSpec((D, D), lambda i: (0, 0)),
            pl.BlockSpec((_CBLK, D), lambda i: (i, 0)),
        ],
        out_specs=pl.BlockSpec((NROW, _CBLK), lambda i: (0, i)),
        out_shape=jax.ShapeDtypeStruct((NROW, VP), jnp.float32),
        scratch_shapes=[pltpu.VMEM((NROW, D), jnp.float32)],
    )(pooled_small, rw, emb)

    yq = y[0]

    kidx = (jnp.zeros((MP, 64), jnp.int32)
            .at[:M, :L].set(keys.astype(jnp.int32))
            .reshape(NW, 16, 128))
    vpad = jnp.zeros((MP * L,), jnp.int32).at[:M * L].set(
        values.astype(jnp.int32).reshape(-1))
    vidx = (jnp.zeros((NW, VTOK), jnp.int32)
            .at[:, :KSEG * L].set(vpad.reshape(NW, KSEG * L))
            .reshape(NW, 13, 128))
    i = jnp.arange(VTOK, dtype=jnp.int32)
    smap = jnp.where(i < KSEG * L, i // L, KSEG).astype(
        jnp.int32).reshape(13, 128)
    zeros = jnp.zeros((VP,), jnp.float32)
    e, w2 = _kv_sc(yq, kidx, vidx, smap, zeros)

    preds = pl.pallas_call(
        _logits_tc,
        grid=(VP // _EBLK,),
        in_specs=[
            pl.BlockSpec((NROW, _EBLK), lambda i: (0, i)),
            pl.BlockSpec((2, _EBLK), lambda i: (0, i)),
            pl.BlockSpec((8, 128), lambda i: (0, 0)),
        ],
        out_specs=pl.BlockSpec((C, 1), lambda i: (0, 0)),
        out_shape=jax.ShapeDtypeStruct((C, 1), jnp.float32),
        scratch_shapes=[pltpu.VMEM((NROW, 1), jnp.float32)],
    )(y, w2, e.reshape(8, 128))
    return preds


# speedup vs baseline: 1.0040x; 1.0011x over previous
import functools

import jax
import jax.numpy as jnp
from jax import lax
from jax.experimental import pallas as pl
from jax.experimental.pallas import tpu as pltpu
from jax.experimental.pallas import tpu_sc as plsc

D = 128
L = 50
V = 100000
VP = 102400
M = 1000
MP = 1024
C = 20
P = 5
NW = 32
KSEG = 32
KIDX = KSEG * 64
VTOK = 1664
NROW = 24

_mesh = plsc.VectorSubcoreMesh(core_axis_name="c", subcore_axis_name="s")


@functools.partial(
    pl.kernel,
    out_type=jax.ShapeDtypeStruct((NW, D), jnp.float32),
    mesh=_mesh,
    scratch_types=[
        pltpu.VMEM((64,), jnp.int32),
        pltpu.VMEM((64, D), jnp.float32),
        pltpu.VMEM((1, D), jnp.float32),
        pltpu.SemaphoreType.DMA,
    ],
    compiler_params=pltpu.CompilerParams(needs_layout_passes=False),
)
def _pool_small_sc(emb_hbm, idx_hbm, out_hbm, idx_v, rows_v, out_v, sem):
    cid = lax.axis_index("c")
    sid = lax.axis_index("s")
    wid = sid * 2 + cid
    pltpu.sync_copy(idx_hbm.at[pl.ds(wid * 64, 64)], idx_v)
    pltpu.async_copy(emb_hbm.at[idx_v], rows_v, sem).wait()

    def row_body(r, acc):
        return tuple(acc[k] + rows_v[r, pl.ds(16 * k, 16)] for k in range(8))

    acc = lax.fori_loop(0, L, row_body,
                        tuple(jnp.zeros((16,), jnp.float32) for _ in range(8)))
    for k in range(8):
        out_v[0, pl.ds(16 * k, 16)] = acc[k] * (1.0 / L)
    pltpu.sync_copy(out_v, out_hbm.at[pl.ds(wid, 1)])


def _qg_tc(pooled_ref, rw_ref, out_ref):
    pooled = pooled_ref[...]
    rw = rw_ref[...]
    enc_cands = pooled[0:C]
    enc_persona = pooled[C:C + P]
    enc_x = pooled[C + P:C + P + 1]
    eps = 1e-6
    dot = jnp.sum(enc_x * enc_persona, axis=1, keepdims=True)
    na = jnp.sqrt(jnp.sum(enc_x * enc_x, axis=1, keepdims=True))
    nb = jnp.sqrt(jnp.sum(enc_persona * enc_persona, axis=1, keepdims=True))
    sim = dot / (jnp.maximum(na, eps) * jnp.maximum(nb, eps))
    m = jnp.max(sim, axis=0, keepdims=True)
    ex = jnp.exp(sim - m)
    ss = ex / jnp.sum(ex, axis=0, keepdims=True)
    test = jnp.dot(ss.T, enc_persona, preferred_element_type=jnp.float32)
    q = jnp.dot(test, rw.T, preferred_element_type=jnp.float32)
    g = jnp.dot(enc_cands, rw, preferred_element_type=jnp.float32)
    out_ref[0:1, :] = q
    out_ref[1:1 + C, :] = g
    out_ref[1 + C:, :] = jnp.zeros((NROW - 1 - C, D), jnp.float32)


_CBLK = 4096

def _table_tc(pooled_ref, rw_ref, emb_ref, y_ref, wall_ref):
    @pl.when(pl.program_id(0) == 0)
    def _():
        _qg_tc(pooled_ref, rw_ref, wall_ref)

    y_ref[...] = jax.lax.dot_general(
        wall_ref[...], emb_ref[...],
        dimension_numbers=(((1,), (1,)), ((), ())),
        preferred_element_type=jnp.float32)


@functools.partial(
    pl.kernel,
    out_type=(jax.ShapeDtypeStruct((MP,), jnp.float32),
              jax.ShapeDtypeStruct((2, VP), jnp.float32)),
    mesh=_mesh,
    scratch_types=[
        pltpu.VMEM((2048,), jnp.int32),
        pltpu.VMEM((2048,), jnp.float32),
        pltpu.VMEM((48,), jnp.float32),
        pltpu.VMEM((13, 128), jnp.int32),
        pltpu.VMEM((13, 128), jnp.int32),
        pltpu.VMEM((13, 128), jnp.float32),
        pltpu.VMEM_SHARED((VP,), jnp.float32),
        pltpu.SemaphoreType.DMA,
    ],
    compiler_params=pltpu.CompilerParams(needs_layout_passes=False),
)
def _kv_sc(yq_hbm, kidx_hbm, vidx_hbm, smap_hbm, zeros_hbm,
           e_hbm, w_hbm, kidx_v, kval_v, e_v, vidx_v, smap_v, u_v, w_sp, sem):
    cid = lax.axis_index("c")
    sid = lax.axis_index("s")
    wid = sid * 2 + cid

    with jax.named_scope("kv_keys_gather"):
        pltpu.sync_copy(kidx_hbm.at[wid], kidx_v)
        pltpu.async_copy(yq_hbm.at[kidx_v], kval_v, sem).wait()

    with jax.named_scope("kv_seg_sums"):
        lanes = lax.iota(jnp.int32, 16)
        for g in range(2):
            base = lanes * 64 + g * 1024
            ssum = jnp.zeros((16,), jnp.float32)
            for t in range(L):
                ssum = ssum + plsc.load_gather(kval_v, [base + t])
            seg_global = wid * KSEG + g * 16 + lanes
            e = jnp.exp(ssum * (1.0 / L))
            e = jnp.where(seg_global < M, e, 0.0)
            e_v[pl.ds(g * 16, 16)] = e
        e_v[pl.ds(32, 16)] = jnp.zeros((16,), jnp.float32)
        pltpu.sync_copy(e_v.at[pl.ds(0, KSEG)],
                        e_hbm.at[pl.ds(wid * KSEG, KSEG)])

    with jax.named_scope("kv_u_build"):
        pltpu.sync_copy(vidx_hbm.at[wid], vidx_v)
        pltpu.sync_copy(smap_hbm, smap_v)
        for j in range(13):
            for t in range(8):
                sm = smap_v[j, pl.ds(16 * t, 16)]
                u_v[j, pl.ds(16 * t, 16)] = plsc.load_gather(e_v, [sm])

    wslc = VP // 16
    with jax.named_scope("kv_w_init"):
        pltpu.sync_copy(zeros_hbm.at[pl.ds(sid * wslc, wslc)],
                        w_sp.at[pl.ds(sid * wslc, wslc)])
        plsc.subcore_barrier()
    with jax.named_scope("kv_scatter"):
        for j in range(13):
            pltpu.sync_copy(u_v.at[j], w_sp.at[vidx_v.at[j]], add=True)
        plsc.subcore_barrier()
    with jax.named_scope("kv_w_out"):
        pltpu.sync_copy(w_sp.at[pl.ds(sid * wslc, wslc)],
                        w_hbm.at[cid, pl.ds(sid * wslc, wslc)])


_EBLK = 6400

def _logits_tc(y_ref, w_ref, e_ref, out_ref, acc_ref):
    i = pl.program_id(0)

    @pl.when(i == 0)
    def _():
        acc_ref[...] = jnp.zeros((NROW, 1), jnp.float32)

    ws = w_ref[0:1, :] + w_ref[1:2, :]
    col = i * _EBLK + jax.lax.broadcasted_iota(jnp.int32, (1, _EBLK), 1)
    yblk = jnp.where(col < V, y_ref[...], 0.0)
    acc_ref[...] += jax.lax.dot_general(
        yblk, ws,
        dimension_numbers=(((1,), (1,)), ((), ())),
        preferred_element_type=jnp.float32)

    @pl.when(i == pl.num_programs(0) - 1)
    def _():
        z = jnp.sum(e_ref[...])
        logits = acc_ref[1:1 + C, :] * (1.0 / (L * z))
        mx = jnp.max(logits, axis=0, keepdims=True)
        ex = jnp.exp(logits - mx)
        out_ref[...] = ex / jnp.sum(ex, axis=0, keepdims=True)


def kernel(xs, candidates, persona, label, keys, values, emb_table, R_W):
    del label
    emb = emb_table.astype(jnp.float32)
    rw = R_W.astype(jnp.float32)

    small = jnp.concatenate([
        candidates.reshape(-1), persona.reshape(-1), xs.reshape(-1),
    ]).astype(jnp.int32).reshape(C + P + 1, L)
    idx_small = (jnp.zeros((NW, 64), jnp.int32)
                 .at[:C + P + 1, :L].set(small).reshape(-1))
    pooled_small = _pool_small_sc(emb, idx_small)

    y = pl.pallas_call(
        _table_tc,
        grid=(VP // _CBLK,),
        in_specs=[
            pl.BlockSpec((NW, D), lambda i: (0, 0)),
            pl.BlockSpec((D, D), lambda i: (0, 0)),
            pl.BlockSpec((_CBLK, D), lambda i: (i, 0)),
        ],
        out_specs=pl.BlockSpec((NROW, _CBLK), lambda i: (0, i)),
        out_shape=jax.ShapeDtypeStruct((NROW, VP), jnp.float32),
        scratch_shapes=[pltpu.VMEM((NROW, D), jnp.float32)],
    )(pooled_small, rw, emb)

    yq = y[0]

    kidx = (jnp.zeros((MP, 64), jnp.int32)
            .at[:M, :L].set(keys.astype(jnp.int32))
            .reshape(NW, 2048))
    vpad = jnp.zeros((MP * L,), jnp.int32).at[:M * L].set(
        values.astype(jnp.int32).reshape(-1))
    vidx = (jnp.zeros((NW, VTOK), jnp.int32)
            .at[:, :KSEG * L].set(vpad.reshape(NW, KSEG * L))
            .reshape(NW, 13, 128))
    i = jnp.arange(VTOK, dtype=jnp.int32)
    smap = jnp.where(i < KSEG * L, i // L, KSEG).astype(
        jnp.int32).reshape(13, 128)
    zeros = jnp.zeros((VP,), jnp.float32)
    e, w2 = _kv_sc(yq, kidx, vidx, smap, zeros)

    preds = pl.pallas_call(
        _logits_tc,
        grid=(VP // _EBLK,),
        in_specs=[
            pl.BlockSpec((NROW, _EBLK), lambda i: (0, i)),
            pl.BlockSpec((2, _EBLK), lambda i: (0, i)),
            pl.BlockSpec((8, 128), lambda i: (0, 0)),
        ],
        out_specs=pl.BlockSpec((C, 1), lambda i: (0, 0)),
        out_shape=jax.ShapeDtypeStruct((C, 1), jnp.float32),
        scratch_shapes=[pltpu.VMEM((NROW, 1), jnp.float32)],
    )(y, w2, e.reshape(8, 128))
    return preds


# speedup vs baseline: 1.5266x; 1.5205x over previous
import functools

import jax
import jax.numpy as jnp
from jax import lax
from jax.experimental import pallas as pl
from jax.experimental.pallas import tpu as pltpu
from jax.experimental.pallas import tpu_sc as plsc

D = 128
L = 50
V = 100000
VP = 102400
M = 1000
MP = 1024
C = 20
P = 5
NW = 32
KSEG = 32
KIDX = KSEG * 64
VTOK = 1664
NROW = 24

_mesh = plsc.VectorSubcoreMesh(core_axis_name="c", subcore_axis_name="s")


@functools.partial(
    pl.kernel,
    out_type=jax.ShapeDtypeStruct((NW, D), jnp.float32),
    mesh=_mesh,
    scratch_types=[
        pltpu.VMEM((64,), jnp.int32),
        pltpu.VMEM((64, D), jnp.float32),
        pltpu.VMEM((1, D), jnp.float32),
        pltpu.SemaphoreType.DMA,
    ],
    compiler_params=pltpu.CompilerParams(needs_layout_passes=False),
)
def _pool_small_sc(emb_hbm, idx_hbm, out_hbm, idx_v, rows_v, out_v, sem):
    cid = lax.axis_index("c")
    sid = lax.axis_index("s")
    wid = sid * 2 + cid
    pltpu.sync_copy(idx_hbm.at[pl.ds(wid * 64, 64)], idx_v)
    pltpu.async_copy(emb_hbm.at[idx_v], rows_v, sem).wait()

    def row_body(r, acc):
        return tuple(acc[k] + rows_v[r, pl.ds(16 * k, 16)] for k in range(8))

    acc = lax.fori_loop(0, L, row_body,
                        tuple(jnp.zeros((16,), jnp.float32) for _ in range(8)))
    for k in range(8):
        out_v[0, pl.ds(16 * k, 16)] = acc[k] * (1.0 / L)
    pltpu.sync_copy(out_v, out_hbm.at[pl.ds(wid, 1)])


def _qg_tc(pooled_ref, rw_ref, out_ref):
    pooled = pooled_ref[...]
    rw = rw_ref[...]
    enc_cands = pooled[0:C]
    enc_persona = pooled[C:C + P]
    enc_x = pooled[C + P:C + P + 1]
    eps = 1e-6
    dot = jnp.sum(enc_x * enc_persona, axis=1, keepdims=True)
    na = jnp.sqrt(jnp.sum(enc_x * enc_x, axis=1, keepdims=True))
    nb = jnp.sqrt(jnp.sum(enc_persona * enc_persona, axis=1, keepdims=True))
    sim = dot / (jnp.maximum(na, eps) * jnp.maximum(nb, eps))
    m = jnp.max(sim, axis=0, keepdims=True)
    ex = jnp.exp(sim - m)
    ss = ex / jnp.sum(ex, axis=0, keepdims=True)
    test = jnp.dot(ss.T, enc_persona, preferred_element_type=jnp.float32)
    q = jnp.dot(test, rw.T, preferred_element_type=jnp.float32)
    g = jnp.dot(enc_cands, rw, preferred_element_type=jnp.float32)
    out_ref[0:1, :] = q
    out_ref[1:1 + C, :] = g
    out_ref[1 + C:, :] = jnp.zeros((NROW - 1 - C, D), jnp.float32)


_CBLK = 4096

def _table_tc(pooled_ref, rw_ref, emb_ref, y_ref, wall_ref):
    @pl.when(pl.program_id(0) == 0)
    def _():
        _qg_tc(pooled_ref, rw_ref, wall_ref)

    y_ref[...] = jax.lax.dot_general(
        wall_ref[...], emb_ref[...],
        dimension_numbers=(((1,), (1,)), ((), ())),
        preferred_element_type=jnp.float32)


@functools.partial(
    pl.kernel,
    out_type=(jax.ShapeDtypeStruct((MP,), jnp.float32),
              jax.ShapeDtypeStruct((2, VP), jnp.float32)),
    mesh=_mesh,
    scratch_types=[
        pltpu.VMEM((VP,), jnp.float32),
        pltpu.VMEM((2048,), jnp.int32),
        pltpu.VMEM((48,), jnp.float32),
        pltpu.VMEM((13, 128), jnp.int32),
        pltpu.VMEM((13, 128), jnp.int32),
        pltpu.VMEM((13, 128), jnp.float32),
        pltpu.VMEM_SHARED((VP,), jnp.float32),
        pltpu.SemaphoreType.DMA,
        pltpu.SemaphoreType.DMA,
    ],
    compiler_params=pltpu.CompilerParams(needs_layout_passes=False),
)
def _kv_sc(yq_hbm, kidx_hbm, vidx_hbm, smap_hbm, zeros_hbm,
           e_hbm, w_hbm, yq_v, kidx_v, e_v, vidx_v, smap_v, u_v, w_sp,
           sem, sem_z):
    cid = lax.axis_index("c")
    sid = lax.axis_index("s")
    wid = sid * 2 + cid
    wslc = VP // 16

    pltpu.async_copy(zeros_hbm.at[pl.ds(sid * wslc, wslc)],
                     w_sp.at[pl.ds(sid * wslc, wslc)], sem_z)

    with jax.named_scope("kv_stage"):
        pltpu.sync_copy(yq_hbm, yq_v)
        pltpu.sync_copy(kidx_hbm.at[wid], kidx_v)
        pltpu.sync_copy(vidx_hbm.at[wid], vidx_v)
        pltpu.sync_copy(smap_hbm, smap_v)

    with jax.named_scope("kv_seg_sums"):
        lanes = lax.iota(jnp.int32, 16)
        for g in range(2):
            base = lanes * 64 + g * 1024
            ssum = jnp.zeros((16,), jnp.float32)
            for t in range(L):
                tok = plsc.load_gather(kidx_v, [base + t])
                ssum = ssum + plsc.load_gather(yq_v, [tok])
            seg_global = wid * KSEG + g * 16 + lanes
            e = jnp.exp(ssum * (1.0 / L))
            e = jnp.where(seg_global < M, e, 0.0)
            e_v[pl.ds(g * 16, 16)] = e
        e_v[pl.ds(32, 16)] = jnp.zeros((16,), jnp.float32)
        pltpu.sync_copy(e_v.at[pl.ds(0, KSEG)],
                        e_hbm.at[pl.ds(wid * KSEG, KSEG)])

    with jax.named_scope("kv_u_build"):
        for j in range(13):
            for t in range(8):
                sm = smap_v[j, pl.ds(16 * t, 16)]
                u_v[j, pl.ds(16 * t, 16)] = plsc.load_gather(e_v, [sm])

    with jax.named_scope("kv_w_init"):
        pltpu.make_async_copy(
            zeros_hbm.at[pl.ds(sid * wslc, wslc)],
            w_sp.at[pl.ds(sid * wslc, wslc)], sem_z).wait()
        plsc.subcore_barrier()
    with jax.named_scope("kv_scatter"):
        for j in range(13):
            pltpu.sync_copy(u_v.at[j], w_sp.at[vidx_v.at[j]], add=True)
        plsc.subcore_barrier()
    with jax.named_scope("kv_w_out"):
        pltpu.sync_copy(w_sp.at[pl.ds(sid * wslc, wslc)],
                        w_hbm.at[cid, pl.ds(sid * wslc, wslc)])


_EBLK = 6400

def _logits_tc(y_ref, w_ref, e_ref, out_ref, acc_ref):
    i = pl.program_id(0)

    @pl.when(i == 0)
    def _():
        acc_ref[...] = jnp.zeros((NROW, 1), jnp.float32)

    ws = w_ref[0:1, :] + w_ref[1:2, :]
    col = i * _EBLK + jax.lax.broadcasted_iota(jnp.int32, (1, _EBLK), 1)
    yblk = jnp.where(col < V, y_ref[...], 0.0)
    acc_ref[...] += jax.lax.dot_general(
        yblk, ws,
        dimension_numbers=(((1,), (1,)), ((), ())),
        preferred_element_type=jnp.float32)

    @pl.when(i == pl.num_programs(0) - 1)
    def _():
        z = jnp.sum(e_ref[...])
        logits = acc_ref[1:1 + C, :] * (1.0 / (L * z))
        mx = jnp.max(logits, axis=0, keepdims=True)
        ex = jnp.exp(logits - mx)
        out_ref[...] = ex / jnp.sum(ex, axis=0, keepdims=True)


def kernel(xs, candidates, persona, label, keys, values, emb_table, R_W):
    del label
    emb = emb_table.astype(jnp.float32)
    rw = R_W.astype(jnp.float32)

    small = jnp.concatenate([
        candidates.reshape(-1), persona.reshape(-1), xs.reshape(-1),
    ]).astype(jnp.int32).reshape(C + P + 1, L)
    idx_small = (jnp.zeros((NW, 64), jnp.int32)
                 .at[:C + P + 1, :L].set(small).reshape(-1))
    pooled_small = _pool_small_sc(emb, idx_small)

    y = pl.pallas_call(
        _table_tc,
        grid=(VP // _CBLK,),
        in_specs=[
            pl.BlockSpec((NW, D), lambda i: (0, 0)),
            pl.BlockSpec((D, D), lambda i: (0, 0)),
            pl.BlockSpec((_CBLK, D), lambda i: (i, 0)),
        ],
        out_specs=pl.BlockSpec((NROW, _CBLK), lambda i: (0, i)),
        out_shape=jax.ShapeDtypeStruct((NROW, VP), jnp.float32),
        scratch_shapes=[pltpu.VMEM((NROW, D), jnp.float32)],
    )(pooled_small, rw, emb)

    yq = y[0]

    kidx = (jnp.zeros((MP, 64), jnp.int32)
            .at[:M, :L].set(keys.astype(jnp.int32))
            .reshape(NW, 2048))
    vpad = jnp.zeros((MP * L,), jnp.int32).at[:M * L].set(
        values.astype(jnp.int32).reshape(-1))
    vidx = (jnp.zeros((NW, VTOK), jnp.int32)
            .at[:, :KSEG * L].set(vpad.reshape(NW, KSEG * L))
            .reshape(NW, 13, 128))
    i = jnp.arange(VTOK, dtype=jnp.int32)
    smap = jnp.where(i < KSEG * L, i // L, KSEG).astype(
        jnp.int32).reshape(13, 128)
    zeros = jnp.zeros((VP,), jnp.float32)
    e, w2 = _kv_sc(yq, kidx, vidx, smap, zeros)

    preds = pl.pallas_call(
        _logits_tc,
        grid=(VP // _EBLK,),
        in_specs=[
            pl.BlockSpec((NROW, _EBLK), lambda i: (0, i)),
            pl.BlockSpec((2, _EBLK), lambda i: (0, i)),
            pl.BlockSpec((8, 128), lambda i: (0, 0)),
        ],
        out_specs=pl.BlockSpec((C, 1), lambda i: (0, 0)),
        out_shape=jax.ShapeDtypeStruct((C, 1), jnp.float32),
        scratch_shapes=[pltpu.VMEM((NROW, 1), jnp.float32)],
    )(y, w2, e.reshape(8, 128))
    return preds


# speedup vs baseline: 1.5352x; 1.0057x over previous
import functools

import jax
import jax.numpy as jnp
from jax import lax
from jax.experimental import pallas as pl
from jax.experimental.pallas import tpu as pltpu
from jax.experimental.pallas import tpu_sc as plsc

D = 128
L = 50
V = 100000
VP = 102400
M = 1000
MP = 1024
C = 20
P = 5
NW = 32
KSEG = 32
KIDX = KSEG * 64
VTOK = 1664
NROW = 24

_mesh = plsc.VectorSubcoreMesh(core_axis_name="c", subcore_axis_name="s")


@functools.partial(
    pl.kernel,
    out_type=jax.ShapeDtypeStruct((NW, D), jnp.float32),
    mesh=_mesh,
    scratch_types=[
        pltpu.VMEM((64,), jnp.int32),
        pltpu.VMEM((64, D), jnp.float32),
        pltpu.VMEM((1, D), jnp.float32),
        pltpu.SemaphoreType.DMA,
    ],
    compiler_params=pltpu.CompilerParams(needs_layout_passes=False),
)
def _pool_small_sc(emb_hbm, idx_hbm, out_hbm, idx_v, rows_v, out_v, sem):
    cid = lax.axis_index("c")
    sid = lax.axis_index("s")
    wid = sid * 2 + cid
    with jax.named_scope("pool_idx"):
        pltpu.sync_copy(idx_hbm.at[pl.ds(wid * 64, 64)], idx_v)
    with jax.named_scope("pool_gather"):
        pltpu.async_copy(emb_hbm.at[idx_v], rows_v, sem).wait()

    with jax.named_scope("pool_accum"):
        def row_body(r, acc):
            return tuple(acc[k] + rows_v[r, pl.ds(16 * k, 16)]
                         for k in range(8))

        acc = lax.fori_loop(
            0, L, row_body,
            tuple(jnp.zeros((16,), jnp.float32) for _ in range(8)))
        for k in range(8):
            out_v[0, pl.ds(16 * k, 16)] = acc[k] * (1.0 / L)
    with jax.named_scope("pool_out"):
        pltpu.sync_copy(out_v, out_hbm.at[pl.ds(wid, 1)])


def _qg_tc(pooled_ref, rw_ref, out_ref):
    pooled = pooled_ref[...]
    rw = rw_ref[...]
    enc_cands = pooled[0:C]
    enc_persona = pooled[C:C + P]
    enc_x = pooled[C + P:C + P + 1]
    eps = 1e-6
    dot = jnp.sum(enc_x * enc_persona, axis=1, keepdims=True)
    na = jnp.sqrt(jnp.sum(enc_x * enc_x, axis=1, keepdims=True))
    nb = jnp.sqrt(jnp.sum(enc_persona * enc_persona, axis=1, keepdims=True))
    sim = dot / (jnp.maximum(na, eps) * jnp.maximum(nb, eps))
    m = jnp.max(sim, axis=0, keepdims=True)
    ex = jnp.exp(sim - m)
    ss = ex / jnp.sum(ex, axis=0, keepdims=True)
    test = jnp.dot(ss.T, enc_persona, preferred_element_type=jnp.float32)
    q = jnp.dot(test, rw.T, preferred_element_type=jnp.float32)
    g = jnp.dot(enc_cands, rw, preferred_element_type=jnp.float32)
    out_ref[0:1, :] = q
    out_ref[1:1 + C, :] = g
    out_ref[1 + C:, :] = jnp.zeros((NROW - 1 - C, D), jnp.float32)


_CBLK = 4096

def _table_tc(pooled_ref, rw_ref, emb_ref, y_ref, wall_ref):
    @pl.when(pl.program_id(0) == 0)
    def _():
        _qg_tc(pooled_ref, rw_ref, wall_ref)

    y_ref[...] = jax.lax.dot_general(
        wall_ref[...], emb_ref[...],
        dimension_numbers=(((1,), (1,)), ((), ())),
        preferred_element_type=jnp.float32)


@functools.partial(
    pl.kernel,
    out_type=(jax.ShapeDtypeStruct((MP,), jnp.float32),
              jax.ShapeDtypeStruct((2, VP), jnp.float32)),
    mesh=_mesh,
    scratch_types=[
        pltpu.VMEM((VP,), jnp.float32),
        pltpu.VMEM((2048,), jnp.int32),
        pltpu.VMEM((48,), jnp.float32),
        pltpu.VMEM((13, 128), jnp.int32),
        pltpu.VMEM((13, 128), jnp.int32),
        pltpu.VMEM((13, 128), jnp.float32),
        pltpu.VMEM_SHARED((VP,), jnp.float32),
        pltpu.SemaphoreType.DMA,
        pltpu.SemaphoreType.DMA,
    ],
    compiler_params=pltpu.CompilerParams(needs_layout_passes=False),
)
def _kv_sc(yq_hbm, kidx_hbm, vidx_hbm, smap_hbm, zeros_hbm,
           e_hbm, w_hbm, yq_v, kidx_v, e_v, vidx_v, smap_v, u_v, w_sp,
           sem, sem_z):
    cid = lax.axis_index("c")
    sid = lax.axis_index("s")
    wid = sid * 2 + cid
    wslc = VP // 16

    pltpu.async_copy(zeros_hbm.at[pl.ds(sid * wslc, wslc)],
                     w_sp.at[pl.ds(sid * wslc, wslc)], sem_z)

    with jax.named_scope("kv_stage"):
        pltpu.sync_copy(yq_hbm, yq_v)
        pltpu.sync_copy(kidx_hbm.at[wid], kidx_v)
        pltpu.sync_copy(vidx_hbm.at[wid], vidx_v)
        pltpu.sync_copy(smap_hbm, smap_v)

    with jax.named_scope("kv_seg_sums"):
        lanes = lax.iota(jnp.int32, 16)
        for g in range(2):
            base = lanes * 64 + g * 1024
            ssum = jnp.zeros((16,), jnp.float32)
            for t in range(L):
                tok = plsc.load_gather(kidx_v, [base + t])
                ssum = ssum + plsc.load_gather(yq_v, [tok])
            seg_global = wid * KSEG + g * 16 + lanes
            e = jnp.exp(ssum * (1.0 / L))
            e = jnp.where(seg_global < M, e, 0.0)
            e_v[pl.ds(g * 16, 16)] = e
        e_v[pl.ds(32, 16)] = jnp.zeros((16,), jnp.float32)
        pltpu.sync_copy(e_v.at[pl.ds(0, KSEG)],
                        e_hbm.at[pl.ds(wid * KSEG, KSEG)])

    with jax.named_scope("kv_u_build"):
        for j in range(13):
            for t in range(8):
                sm = smap_v[j, pl.ds(16 * t, 16)]
                u_v[j, pl.ds(16 * t, 16)] = plsc.load_gather(e_v, [sm])

    with jax.named_scope("kv_w_init"):
        pltpu.make_async_copy(
            zeros_hbm.at[pl.ds(sid * wslc, wslc)],
            w_sp.at[pl.ds(sid * wslc, wslc)], sem_z).wait()
        plsc.subcore_barrier()
    with jax.named_scope("kv_scatter"):
        for j in range(13):
            pltpu.sync_copy(u_v.at[j], w_sp.at[vidx_v.at[j]], add=True)
        plsc.subcore_barrier()
    with jax.named_scope("kv_w_out"):
        pltpu.sync_copy(w_sp.at[pl.ds(sid * wslc, wslc)],
                        w_hbm.at[cid, pl.ds(sid * wslc, wslc)])


_EBLK = 6400

def _logits_tc(y_ref, w_ref, e_ref, out_ref, acc_ref):
    i = pl.program_id(0)

    @pl.when(i == 0)
    def _():
        acc_ref[...] = jnp.zeros((NROW, 1), jnp.float32)

    ws = w_ref[0:1, :] + w_ref[1:2, :]
    col = i * _EBLK + jax.lax.broadcasted_iota(jnp.int32, (1, _EBLK), 1)
    yblk = jnp.where(col < V, y_ref[...], 0.0)
    acc_ref[...] += jax.lax.dot_general(
        yblk, ws,
        dimension_numbers=(((1,), (1,)), ((), ())),
        preferred_element_type=jnp.float32)

    @pl.when(i == pl.num_programs(0) - 1)
    def _():
        z = jnp.sum(e_ref[...])
        logits = acc_ref[1:1 + C, :] * (1.0 / (L * z))
        mx = jnp.max(logits, axis=0, keepdims=True)
        ex = jnp.exp(logits - mx)
        out_ref[...] = ex / jnp.sum(ex, axis=0, keepdims=True)


def kernel(xs, candidates, persona, label, keys, values, emb_table, R_W):
    del label
    emb = emb_table.astype(jnp.float32)
    rw = R_W.astype(jnp.float32)

    small = jnp.concatenate([
        candidates.reshape(-1), persona.reshape(-1), xs.reshape(-1),
    ]).astype(jnp.int32).reshape(C + P + 1, L)
    idx_small = (jnp.zeros((NW, 64), jnp.int32)
                 .at[:C + P + 1, :L].set(small).reshape(-1))
    pooled_small = _pool_small_sc(emb, idx_small)

    y = pl.pallas_call(
        _table_tc,
        grid=(VP // _CBLK,),
        in_specs=[
            pl.BlockSpec((NW, D), lambda i: (0, 0)),
            pl.BlockSpec((D, D), lambda i: (0, 0)),
            pl.BlockSpec((_CBLK, D), lambda i: (i, 0)),
        ],
        out_specs=pl.BlockSpec((NROW, _CBLK), lambda i: (0, i)),
        out_shape=jax.ShapeDtypeStruct((NROW, VP), jnp.float32),
        scratch_shapes=[pltpu.VMEM((NROW, D), jnp.float32)],
    )(pooled_small, rw, emb)

    yq = y[0]

    kidx = (jnp.zeros((MP, 64), jnp.int32)
            .at[:M, :L].set(keys.astype(jnp.int32))
            .reshape(NW, 2048))
    vpad = jnp.zeros((MP * L,), jnp.int32).at[:M * L].set(
        values.astype(jnp.int32).reshape(-1))
    vidx = (jnp.zeros((NW, VTOK), jnp.int32)
            .at[:, :KSEG * L].set(vpad.reshape(NW, KSEG * L))
            .reshape(NW, 13, 128))
    i = jnp.arange(VTOK, dtype=jnp.int32)
    smap = jnp.where(i < KSEG * L, i // L, KSEG).astype(
        jnp.int32).reshape(13, 128)
    zeros = jnp.zeros((VP,), jnp.float32)
    e, w2 = _kv_sc(yq, kidx, vidx, smap, zeros)

    preds = pl.pallas_call(
        _logits_tc,
        grid=(VP // _EBLK,),
        in_specs=[
            pl.BlockSpec((NROW, _EBLK), lambda i: (0, i)),
            pl.BlockSpec((2, _EBLK), lambda i: (0, i)),
            pl.BlockSpec((8, 128), lambda i: (0, 0)),
        ],
        out_specs=pl.BlockSpec((C, 1), lambda i: (0, 0)),
        out_shape=jax.ShapeDtypeStruct((C, 1), jnp.float32),
        scratch_shapes=[pltpu.VMEM((NROW, 1), jnp.float32)],
    )(y, w2, e.reshape(8, 128))
    return preds


# speedup vs baseline: 1.7633x; 1.1486x over previous
import functools

import jax
import jax.numpy as jnp
from jax import lax
from jax.experimental import pallas as pl
from jax.experimental.pallas import tpu as pltpu
from jax.experimental.pallas import tpu_sc as plsc

D = 128
L = 50
V = 100000
VP = 102400
M = 1000
MP = 1024
C = 20
P = 5
NW = 32
KSEG = 32
KIDX = KSEG * 64
VTOK = 1664
NROW = 24

_mesh = plsc.VectorSubcoreMesh(core_axis_name="c", subcore_axis_name="s")


@functools.partial(
    pl.kernel,
    out_type=jax.ShapeDtypeStruct((NW, D), jnp.float32),
    mesh=_mesh,
    scratch_types=[
        pltpu.VMEM((56,), jnp.int32),
        pltpu.VMEM((56, D), jnp.float32),
        pltpu.VMEM((1, D), jnp.float32),
        pltpu.SemaphoreType.DMA,
    ],
    compiler_params=pltpu.CompilerParams(needs_layout_passes=False),
)
def _pool_small_sc(emb_hbm, idx_hbm, out_hbm, idx_v, rows_v, out_v, sem):
    cid = lax.axis_index("c")
    sid = lax.axis_index("s")
    wid = sid * 2 + cid
    pltpu.sync_copy(idx_hbm.at[pl.ds(wid * 64, 56)], idx_v)
    pltpu.async_copy(emb_hbm.at[idx_v], rows_v, sem).wait()

    def row_body(r, acc):
        return tuple(acc[k] + rows_v[r, pl.ds(16 * k, 16)]
                     for k in range(8))

    acc = lax.fori_loop(
        0, L, row_body,
        tuple(jnp.zeros((16,), jnp.float32) for _ in range(8)))
    for k in range(8):
        out_v[0, pl.ds(16 * k, 16)] = acc[k] * (1.0 / L)
    pltpu.sync_copy(out_v, out_hbm.at[pl.ds(wid, 1)])


def _qg_tc(pooled_ref, rw_ref, out_ref):
    pooled = pooled_ref[...]
    rw = rw_ref[...]
    enc_cands = pooled[0:C]
    enc_persona = pooled[C:C + P]
    enc_x = pooled[C + P:C + P + 1]
    eps = 1e-6
    dot = jnp.sum(enc_x * enc_persona, axis=1, keepdims=True)
    na = jnp.sqrt(jnp.sum(enc_x * enc_x, axis=1, keepdims=True))
    nb = jnp.sqrt(jnp.sum(enc_persona * enc_persona, axis=1, keepdims=True))
    sim = dot / (jnp.maximum(na, eps) * jnp.maximum(nb, eps))
    m = jnp.max(sim, axis=0, keepdims=True)
    ex = jnp.exp(sim - m)
    ss = ex / jnp.sum(ex, axis=0, keepdims=True)
    test = jnp.dot(ss.T, enc_persona, preferred_element_type=jnp.float32)
    q = jnp.dot(test, rw.T, preferred_element_type=jnp.float32)
    g = jnp.dot(enc_cands, rw, preferred_element_type=jnp.float32)
    out_ref[0:1, :] = q
    out_ref[1:1 + C, :] = g
    out_ref[1 + C:, :] = jnp.zeros((NROW - 1 - C, D), jnp.float32)


_CBLK = 4096

def _table_tc(pooled_ref, rw_ref, emb_ref, y_ref, wall_ref):
    @pl.when(pl.program_id(0) == 0)
    def _():
        _qg_tc(pooled_ref, rw_ref, wall_ref)

    y_ref[...] = jax.lax.dot_general(
        wall_ref[...], emb_ref[...],
        dimension_numbers=(((1,), (1,)), ((), ())),
        preferred_element_type=jnp.float32)


@functools.partial(
    pl.kernel,
    out_type=(jax.ShapeDtypeStruct((MP,), jnp.float32),
              jax.ShapeDtypeStruct((2, VP), jnp.float32)),
    mesh=_mesh,
    scratch_types=[
        pltpu.VMEM((2048,), jnp.int32),
        pltpu.VMEM((2048,), jnp.float32),
        pltpu.VMEM((48,), jnp.float32),
        pltpu.VMEM((13, 128), jnp.int32),
        pltpu.VMEM((13, 128), jnp.int32),
        pltpu.VMEM((13, 128), jnp.float32),
        pltpu.VMEM_SHARED((VP,), jnp.float32),
        pltpu.VMEM_SHARED((VP,), jnp.float32),
        pltpu.SemaphoreType.DMA,
        pltpu.SemaphoreType.DMA,
    ],
    compiler_params=pltpu.CompilerParams(needs_layout_passes=False),
)
def _kv_sc(yq_hbm, kidx_hbm, vidx_hbm, smap_hbm, zeros_hbm,
           e_hbm, w_hbm, kidx_v, kval_v, e_v, vidx_v, smap_v, u_v,
           yq_sp, w_sp, sem, sem_z):
    cid = lax.axis_index("c")
    sid = lax.axis_index("s")
    wid = sid * 2 + cid
    wslc = VP // 16

    pltpu.async_copy(zeros_hbm.at[pl.ds(sid * wslc, wslc)],
                     w_sp.at[pl.ds(sid * wslc, wslc)], sem_z)

    with jax.named_scope("kv_stage"):
        pltpu.sync_copy(yq_hbm.at[pl.ds(sid * wslc, wslc)],
                        yq_sp.at[pl.ds(sid * wslc, wslc)])
        pltpu.sync_copy(kidx_hbm.at[wid], kidx_v)
        pltpu.sync_copy(vidx_hbm.at[wid], vidx_v)
        pltpu.sync_copy(smap_hbm, smap_v)
        plsc.subcore_barrier()

    with jax.named_scope("kv_keys_gather"):
        pltpu.async_copy(yq_sp.at[kidx_v], kval_v, sem).wait()
    with jax.named_scope("kv_seg_sums"):
        lanes = lax.iota(jnp.int32, 16)
        for g in range(2):
            base = lanes * 64 + g * 1024
            ssum = jnp.zeros((16,), jnp.float32)
            for t in range(L):
                ssum = ssum + plsc.load_gather(kval_v, [base + t])
            seg_global = wid * KSEG + g * 16 + lanes
            e = jnp.exp(ssum * (1.0 / L))
            e = jnp.where(seg_global < M, e, 0.0)
            e_v[pl.ds(g * 16, 16)] = e
        e_v[pl.ds(32, 16)] = jnp.zeros((16,), jnp.float32)
        pltpu.sync_copy(e_v.at[pl.ds(0, KSEG)],
                        e_hbm.at[pl.ds(wid * KSEG, KSEG)])

    with jax.named_scope("kv_u_build"):
        for j in range(13):
            for t in range(8):
                sm = smap_v[j, pl.ds(16 * t, 16)]
                u_v[j, pl.ds(16 * t, 16)] = plsc.load_gather(e_v, [sm])

    with jax.named_scope("kv_w_init"):
        pltpu.make_async_copy(
            zeros_hbm.at[pl.ds(sid * wslc, wslc)],
            w_sp.at[pl.ds(sid * wslc, wslc)], sem_z).wait()
        plsc.subcore_barrier()
    with jax.named_scope("kv_scatter"):
        for j in range(13):
            pltpu.sync_copy(u_v.at[j], w_sp.at[vidx_v.at[j]], add=True)
        plsc.subcore_barrier()
    with jax.named_scope("kv_w_out"):
        pltpu.sync_copy(w_sp.at[pl.ds(sid * wslc, wslc)],
                        w_hbm.at[cid, pl.ds(sid * wslc, wslc)])


_EBLK = 6400

def _logits_tc(y_ref, w_ref, e_ref, out_ref, acc_ref):
    i = pl.program_id(0)

    @pl.when(i == 0)
    def _():
        acc_ref[...] = jnp.zeros((NROW, 1), jnp.float32)

    ws = w_ref[0:1, :] + w_ref[1:2, :]
    col = i * _EBLK + jax.lax.broadcasted_iota(jnp.int32, (1, _EBLK), 1)
    yblk = jnp.where(col < V, y_ref[...], 0.0)
    acc_ref[...] += jax.lax.dot_general(
        yblk, ws,
        dimension_numbers=(((1,), (1,)), ((), ())),
        preferred_element_type=jnp.float32)

    @pl.when(i == pl.num_programs(0) - 1)
    def _():
        z = jnp.sum(e_ref[...])
        logits = acc_ref[1:1 + C, :] * (1.0 / (L * z))
        mx = jnp.max(logits, axis=0, keepdims=True)
        ex = jnp.exp(logits - mx)
        out_ref[...] = ex / jnp.sum(ex, axis=0, keepdims=True)


def kernel(xs, candidates, persona, label, keys, values, emb_table, R_W):
    del label
    emb = emb_table.astype(jnp.float32)
    rw = R_W.astype(jnp.float32)

    small = jnp.concatenate([
        candidates.reshape(-1), persona.reshape(-1), xs.reshape(-1),
    ]).astype(jnp.int32).reshape(C + P + 1, L)
    idx_small = (jnp.zeros((NW, 64), jnp.int32)
                 .at[:C + P + 1, :L].set(small).reshape(-1))
    pooled_small = _pool_small_sc(emb, idx_small)

    y = pl.pallas_call(
        _table_tc,
        grid=(VP // _CBLK,),
        in_specs=[
            pl.BlockSpec((NW, D), lambda i: (0, 0)),
            pl.BlockSpec((D, D), lambda i: (0, 0)),
            pl.BlockSpec((_CBLK, D), lambda i: (i, 0)),
        ],
        out_specs=pl.BlockSpec((NROW, _CBLK), lambda i: (0, i)),
        out_shape=jax.ShapeDtypeStruct((NROW, VP), jnp.float32),
        scratch_shapes=[pltpu.VMEM((NROW, D), jnp.float32)],
    )(pooled_small, rw, emb)

    yq = y[0]

    kidx = (jnp.zeros((MP, 64), jnp.int32)
            .at[:M, :L].set(keys.astype(jnp.int32))
            .reshape(NW, 2048))
    vpad = jnp.zeros((MP * L,), jnp.int32).at[:M * L].set(
        values.astype(jnp.int32).reshape(-1))
    vidx = (jnp.zeros((NW, VTOK), jnp.int32)
            .at[:, :KSEG * L].set(vpad.reshape(NW, KSEG * L))
            .reshape(NW, 13, 128))
    i = jnp.arange(VTOK, dtype=jnp.int32)
    smap = jnp.where(i < KSEG * L, i // L, KSEG).astype(
        jnp.int32).reshape(13, 128)
    zeros = jnp.zeros((VP,), jnp.float32)
    e, w2 = _kv_sc(yq, kidx, vidx, smap, zeros)

    preds = pl.pallas_call(
        _logits_tc,
        grid=(VP // _EBLK,),
        in_specs=[
            pl.BlockSpec((NROW, _EBLK), lambda i: (0, i)),
            pl.BlockSpec((2, _EBLK), lambda i: (0, i)),
            pl.BlockSpec((8, 128), lambda i: (0, 0)),
        ],
        out_specs=pl.BlockSpec((C, 1), lambda i: (0, 0)),
        out_shape=jax.ShapeDtypeStruct((C, 1), jnp.float32),
        scratch_shapes=[pltpu.VMEM((NROW, 1), jnp.float32)],
    )(y, w2, e.reshape(8, 128))
    return preds


# speedup vs baseline: 1.8074x; 1.0250x over previous
import functools

import jax
import jax.numpy as jnp
from jax import lax
from jax.experimental import pallas as pl
from jax.experimental.pallas import tpu as pltpu
from jax.experimental.pallas import tpu_sc as plsc

D = 128
L = 50
V = 100000
VP = 102400
M = 1000
MP = 1024
C = 20
P = 5
NW = 32
KSEG = 32
KIDX = KSEG * 64
VTOK = 1664
NROW = 24

_mesh = plsc.VectorSubcoreMesh(core_axis_name="c", subcore_axis_name="s")


@functools.partial(
    pl.kernel,
    out_type=jax.ShapeDtypeStruct((NW, D), jnp.float32),
    mesh=_mesh,
    scratch_types=[
        pltpu.VMEM((56,), jnp.int32),
        pltpu.VMEM((56, D), jnp.float32),
        pltpu.VMEM((1, D), jnp.float32),
        pltpu.SemaphoreType.DMA,
    ],
    compiler_params=pltpu.CompilerParams(needs_layout_passes=False),
)
def _pool_small_sc(emb_hbm, idx_hbm, out_hbm, idx_v, rows_v, out_v, sem):
    cid = lax.axis_index("c")
    sid = lax.axis_index("s")
    wid = sid * 2 + cid
    pltpu.sync_copy(idx_hbm.at[pl.ds(wid * 64, 56)], idx_v)
    pltpu.async_copy(emb_hbm.at[idx_v], rows_v, sem).wait()

    def row_body(r, acc):
        return tuple(acc[k] + rows_v[r, pl.ds(16 * k, 16)]
                     for k in range(8))

    acc = lax.fori_loop(
        0, L, row_body,
        tuple(jnp.zeros((16,), jnp.float32) for _ in range(8)))
    for k in range(8):
        out_v[0, pl.ds(16 * k, 16)] = acc[k] * (1.0 / L)
    pltpu.sync_copy(out_v, out_hbm.at[pl.ds(wid, 1)])


def _qg_tc(pooled_ref, rw_ref, out_ref):
    pooled = pooled_ref[...]
    rw = rw_ref[...]
    enc_cands = pooled[0:C]
    enc_persona = pooled[C:C + P]
    enc_x = pooled[C + P:C + P + 1]
    eps = 1e-6
    dot = jnp.sum(enc_x * enc_persona, axis=1, keepdims=True)
    na = jnp.sqrt(jnp.sum(enc_x * enc_x, axis=1, keepdims=True))
    nb = jnp.sqrt(jnp.sum(enc_persona * enc_persona, axis=1, keepdims=True))
    sim = dot / (jnp.maximum(na, eps) * jnp.maximum(nb, eps))
    m = jnp.max(sim, axis=0, keepdims=True)
    ex = jnp.exp(sim - m)
    ss = ex / jnp.sum(ex, axis=0, keepdims=True)
    test = jnp.dot(ss.T, enc_persona, preferred_element_type=jnp.float32)
    q = jnp.dot(test, rw.T, preferred_element_type=jnp.float32)
    g = jnp.dot(enc_cands, rw, preferred_element_type=jnp.float32)
    out_ref[0:1, :] = q
    out_ref[1:1 + C, :] = g
    out_ref[1 + C:, :] = jnp.zeros((NROW - 1 - C, D), jnp.float32)


_CBLK = 4096

def _table_tc(pooled_ref, rw_ref, emb_ref, y_ref, wall_ref):
    @pl.when(pl.program_id(0) == 0)
    def _():
        _qg_tc(pooled_ref, rw_ref, wall_ref)

    y_ref[...] = jax.lax.dot_general(
        wall_ref[...], emb_ref[...],
        dimension_numbers=(((1,), (1,)), ((), ())),
        preferred_element_type=jnp.float32).astype(jnp.bfloat16)


@functools.partial(
    pl.kernel,
    out_type=(jax.ShapeDtypeStruct((MP,), jnp.float32),
              jax.ShapeDtypeStruct((2, VP), jnp.float32)),
    mesh=_mesh,
    scratch_types=[
        pltpu.VMEM((2048,), jnp.int32),
        pltpu.VMEM((2048,), jnp.float32),
        pltpu.VMEM((48,), jnp.float32),
        pltpu.VMEM((13, 128), jnp.int32),
        pltpu.VMEM((13, 128), jnp.int32),
        pltpu.VMEM((13, 128), jnp.float32),
        pltpu.VMEM_SHARED((VP,), jnp.float32),
        pltpu.VMEM_SHARED((VP,), jnp.float32),
        pltpu.SemaphoreType.DMA,
        pltpu.SemaphoreType.DMA,
    ],
    compiler_params=pltpu.CompilerParams(needs_layout_passes=False),
)
def _kv_sc(yq_hbm, kidx_hbm, vidx_hbm, smap_hbm, zeros_hbm,
           e_hbm, w_hbm, kidx_v, kval_v, e_v, vidx_v, smap_v, u_v,
           yq_sp, w_sp, sem, sem_z):
    cid = lax.axis_index("c")
    sid = lax.axis_index("s")
    wid = sid * 2 + cid
    wslc = VP // 16

    pltpu.async_copy(zeros_hbm.at[pl.ds(sid * wslc, wslc)],
                     w_sp.at[pl.ds(sid * wslc, wslc)], sem_z)

    with jax.named_scope("kv_stage"):
        pltpu.sync_copy(yq_hbm.at[pl.ds(sid * wslc, wslc)],
                        yq_sp.at[pl.ds(sid * wslc, wslc)])
        pltpu.sync_copy(kidx_hbm.at[wid], kidx_v)
        pltpu.sync_copy(vidx_hbm.at[wid], vidx_v)
        pltpu.sync_copy(smap_hbm, smap_v)
        plsc.subcore_barrier()

    with jax.named_scope("kv_keys_gather"):
        pltpu.async_copy(yq_sp.at[kidx_v], kval_v, sem).wait()
    with jax.named_scope("kv_seg_sums"):
        lanes = lax.iota(jnp.int32, 16)
        for g in range(2):
            base = lanes * 64 + g * 1024
            ssum = jnp.zeros((16,), jnp.float32)
            for t in range(L):
                ssum = ssum + plsc.load_gather(kval_v, [base + t])
            seg_global = wid * KSEG + g * 16 + lanes
            e = jnp.exp(ssum * (1.0 / L))
            e = jnp.where(seg_global < M, e, 0.0)
            e_v[pl.ds(g * 16, 16)] = e
        e_v[pl.ds(32, 16)] = jnp.zeros((16,), jnp.float32)
        pltpu.sync_copy(e_v.at[pl.ds(0, KSEG)],
                        e_hbm.at[pl.ds(wid * KSEG, KSEG)])

    with jax.named_scope("kv_u_build"):
        for j in range(13):
            for t in range(8):
                sm = smap_v[j, pl.ds(16 * t, 16)]
                u_v[j, pl.ds(16 * t, 16)] = plsc.load_gather(e_v, [sm])

    with jax.named_scope("kv_w_init"):
        pltpu.make_async_copy(
            zeros_hbm.at[pl.ds(sid * wslc, wslc)],
            w_sp.at[pl.ds(sid * wslc, wslc)], sem_z).wait()
        plsc.subcore_barrier()
    with jax.named_scope("kv_scatter"):
        for j in range(13):
            pltpu.sync_copy(u_v.at[j], w_sp.at[vidx_v.at[j]], add=True)
        plsc.subcore_barrier()
    with jax.named_scope("kv_w_out"):
        pltpu.sync_copy(w_sp.at[pl.ds(sid * wslc, wslc)],
                        w_hbm.at[cid, pl.ds(sid * wslc, wslc)])


_EBLK = 6400

def _logits_tc(y_ref, w_ref, e_ref, out_ref, acc_ref):
    i = pl.program_id(0)

    @pl.when(i == 0)
    def _():
        acc_ref[...] = jnp.zeros((NROW, 1), jnp.float32)

    ws = w_ref[0:1, :] + w_ref[1:2, :]
    col = i * _EBLK + jax.lax.broadcasted_iota(jnp.int32, (1, _EBLK), 1)
    yblk = jnp.where(col < V, y_ref[...].astype(jnp.float32), 0.0)
    acc_ref[...] += jax.lax.dot_general(
        yblk, ws,
        dimension_numbers=(((1,), (1,)), ((), ())),
        preferred_element_type=jnp.float32)

    @pl.when(i == pl.num_programs(0) - 1)
    def _():
        z = jnp.sum(e_ref[...])
        logits = acc_ref[1:1 + C, :] * (1.0 / (L * z))
        mx = jnp.max(logits, axis=0, keepdims=True)
        ex = jnp.exp(logits - mx)
        out_ref[...] = ex / jnp.sum(ex, axis=0, keepdims=True)


def kernel(xs, candidates, persona, label, keys, values, emb_table, R_W):
    del label
    emb = emb_table.astype(jnp.float32)
    rw = R_W.astype(jnp.float32)

    small = jnp.concatenate([
        candidates.reshape(-1), persona.reshape(-1), xs.reshape(-1),
    ]).astype(jnp.int32).reshape(C + P + 1, L)
    idx_small = (jnp.zeros((NW, 64), jnp.int32)
                 .at[:C + P + 1, :L].set(small).reshape(-1))
    pooled_small = _pool_small_sc(emb, idx_small)

    y = pl.pallas_call(
        _table_tc,
        grid=(VP // _CBLK,),
        in_specs=[
            pl.BlockSpec((NW, D), lambda i: (0, 0)),
            pl.BlockSpec((D, D), lambda i: (0, 0)),
            pl.BlockSpec((_CBLK, D), lambda i: (i, 0)),
        ],
        out_specs=pl.BlockSpec((NROW, _CBLK), lambda i: (0, i)),
        out_shape=jax.ShapeDtypeStruct((NROW, VP), jnp.bfloat16),
        scratch_shapes=[pltpu.VMEM((NROW, D), jnp.float32)],
    )(pooled_small, rw, emb)

    yq = y[0].astype(jnp.float32)

    kidx = (jnp.zeros((MP, 64), jnp.int32)
            .at[:M, :L].set(keys.astype(jnp.int32))
            .reshape(NW, 2048))
    vpad = jnp.zeros((MP * L,), jnp.int32).at[:M * L].set(
        values.astype(jnp.int32).reshape(-1))
    vidx = (jnp.zeros((NW, VTOK), jnp.int32)
            .at[:, :KSEG * L].set(vpad.reshape(NW, KSEG * L))
            .reshape(NW, 13, 128))
    i = jnp.arange(VTOK, dtype=jnp.int32)
    smap = jnp.where(i < KSEG * L, i // L, KSEG).astype(
        jnp.int32).reshape(13, 128)
    zeros = jnp.zeros((VP,), jnp.float32)
    e, w2 = _kv_sc(yq, kidx, vidx, smap, zeros)

    preds = pl.pallas_call(
        _logits_tc,
        grid=(VP // _EBLK,),
        in_specs=[
            pl.BlockSpec((NROW, _EBLK), lambda i: (0, i)),
            pl.BlockSpec((2, _EBLK), lambda i: (0, i)),
            pl.BlockSpec((8, 128), lambda i: (0, 0)),
        ],
        out_specs=pl.BlockSpec((C, 1), lambda i: (0, 0)),
        out_shape=jax.ShapeDtypeStruct((C, 1), jnp.float32),
        scratch_shapes=[pltpu.VMEM((NROW, 1), jnp.float32)],
    )(y, w2, e.reshape(8, 128))
    return preds
